# Initial kernel scaffold; baseline (speedup 1.0000x reference)
import jax
import jax.numpy as jnp
from jax.experimental import pallas as pl


def kernel(output, bank):
    raise NotImplementedError("write your pallas kernel here")


# speedup vs baseline: 1.3466x; 1.3466x over previous
import jax
import jax.numpy as jnp
from jax.experimental import pallas as pl

_SIZE = 65536
_DIM = 128
_BATCH = 1024
_BLK = 4096
_GRID = _SIZE // _BLK


def _body(out_in_ref, bank_ref, out_out_ref, snap_ref, new_ref):
    i = pl.program_id(0)
    b = bank_ref[...]
    snap_ref[...] = b
    new_ref[...] = b

    @pl.when(i == 0)
    def _():
        out_out_ref[...] = out_in_ref[...]
        new_ref[:, :_BATCH] = jnp.transpose(out_in_ref[...])


def kernel(output, bank):
    out_shapes = (
        jax.ShapeDtypeStruct((_BATCH, _DIM), output.dtype),
        jax.ShapeDtypeStruct((_DIM, _SIZE), bank.dtype),
        jax.ShapeDtypeStruct((_DIM, _SIZE), bank.dtype),
    )
    out, snap, new = pl.pallas_call(
        _body,
        grid=(_GRID,),
        in_specs=[
            pl.BlockSpec((_BATCH, _DIM), lambda i: (0, 0)),
            pl.BlockSpec((_DIM, _BLK), lambda i: (0, i)),
        ],
        out_specs=[
            pl.BlockSpec((_BATCH, _DIM), lambda i: (0, 0)),
            pl.BlockSpec((_DIM, _BLK), lambda i: (0, i)),
            pl.BlockSpec((_DIM, _BLK), lambda i: (0, i)),
        ],
        out_shape=out_shapes,
    )(output, bank)
    return (out, snap, new)


# speedup vs baseline: 1.4693x; 1.0912x over previous
import jax
import jax.numpy as jnp
from jax.experimental import pallas as pl

_SIZE = 65536
_DIM = 128
_BATCH = 1024
_BLK = 8192
_GRID = _SIZE // _BLK


def _body(out_in_ref, bank_ref, out_out_ref, snap_ref, new_ref):
    i = pl.program_id(0)
    b = bank_ref[...]
    snap_ref[...] = b
    new_ref[...] = b

    @pl.when(i == 0)
    def _():
        out_out_ref[...] = out_in_ref[...]
        new_ref[:, :_BATCH] = jnp.transpose(out_in_ref[...])


def kernel(output, bank):
    out_shapes = (
        jax.ShapeDtypeStruct((_BATCH, _DIM), output.dtype),
        jax.ShapeDtypeStruct((_DIM, _SIZE), bank.dtype),
        jax.ShapeDtypeStruct((_DIM, _SIZE), bank.dtype),
    )
    out, snap, new = pl.pallas_call(
        _body,
        grid=(_GRID,),
        in_specs=[
            pl.BlockSpec((_BATCH, _DIM), lambda i: (0, 0)),
            pl.BlockSpec((_DIM, _BLK), lambda i: (0, i)),
        ],
        out_specs=[
            pl.BlockSpec((_BATCH, _DIM), lambda i: (0, 0)),
            pl.BlockSpec((_DIM, _BLK), lambda i: (0, i)),
            pl.BlockSpec((_DIM, _BLK), lambda i: (0, i)),
        ],
        out_shape=out_shapes,
    )(output, bank)
    return (out, snap, new)


# speedup vs baseline: 1.5530x; 1.0569x over previous
import jax
import jax.numpy as jnp
from jax.experimental import pallas as pl

_SIZE = 65536
_DIM = 128
_BATCH = 1024
_BLK = 16384
_GRID = _SIZE // _BLK


def _body(out_in_ref, bank_ref, out_out_ref, snap_ref, new_ref):
    i = pl.program_id(0)
    b = bank_ref[...]
    snap_ref[...] = b
    new_ref[...] = b

    @pl.when(i == 0)
    def _():
        out_out_ref[...] = out_in_ref[...]
        new_ref[:, :_BATCH] = jnp.transpose(out_in_ref[...])


def kernel(output, bank):
    out_shapes = (
        jax.ShapeDtypeStruct((_BATCH, _DIM), output.dtype),
        jax.ShapeDtypeStruct((_DIM, _SIZE), bank.dtype),
        jax.ShapeDtypeStruct((_DIM, _SIZE), bank.dtype),
    )
    out, snap, new = pl.pallas_call(
        _body,
        grid=(_GRID,),
        in_specs=[
            pl.BlockSpec((_BATCH, _DIM), lambda i: (0, 0)),
            pl.BlockSpec((_DIM, _BLK), lambda i: (0, i)),
        ],
        out_specs=[
            pl.BlockSpec((_BATCH, _DIM), lambda i: (0, 0)),
            pl.BlockSpec((_DIM, _BLK), lambda i: (0, i)),
            pl.BlockSpec((_DIM, _BLK), lambda i: (0, i)),
        ],
        out_shape=out_shapes,
    )(output, bank)
    return (out, snap, new)
